# baseline (speedup 1.0000x reference)
import jax
import jax.numpy as jnp
from jax import lax
from jax.experimental import pallas as pl
from jax.experimental.pallas import tpu as pltpu


def _tile_level(t, n_rows, row_len, modulus):
    pad = modulus - t.shape[-1]
    tp = jnp.pad(t, [(0, 0)] * (t.ndim - 1) + [(0, pad)])
    tiled = jnp.tile(tp[..., None, :],
                     tuple([1] * (t.ndim - 1)) + (n_rows, 1))
    flat = tiled.reshape(t.shape[:-1] + (n_rows * modulus,))
    flat = flat[..., : n_rows * row_len]
    return flat.reshape(t.shape[:-1] + (n_rows, row_len))


def _build_w1(conv1_w):
    k = conv1_w.reshape(10, 5, 5).astype(jnp.bfloat16)
    k = jnp.pad(k, ((0, 0), (0, 0), (0, 23))).reshape(10, 140)
    s = _tile_level(k, 2, 784, 785)
    s = _tile_level(s, 2, 784, 812)
    s = _tile_level(s, 12, 784, 840)
    s = _tile_level(s, 12, 784, 786)
    return [s[:, e, d].reshape(1440, 784)
            for e in range(2) for d in range(2)]


def _build_w2(conv2_w):
    k = conv2_w.astype(jnp.bfloat16)
    k = jnp.pad(k, ((0, 0), (0, 0), (0, 7), (0, 7)))
    k = k.reshape(20, 1440)
    s = _tile_level(k, 2, 1440, 1441)
    s = _tile_level(s, 2, 1440, 1452)
    s = _tile_level(s, 4, 1440, 1464)
    s = _tile_level(s, 4, 1440, 1442)
    return [s[:, e, d].reshape(320, 1440)
            for e in range(2) for d in range(2)]


_DN = (((1,), (1,)), ((), ()))


def _lenet_kernel(x_ref, w1a_ref, w1b_ref, w1c_ref, w1d_ref, b1_ref,
                  w2a_ref, w2b_ref, w2c_ref, w2d_ref, b2_ref,
                  wf1_ref, bf1_ref, wf2_ref, bf2_ref, o_ref):
    x = x_ref[...].astype(jnp.bfloat16)
    m1 = None
    for wref in (w1a_ref, w1b_ref, w1c_ref, w1d_ref):
        y = lax.dot_general(x, wref[...], _DN,
                            preferred_element_type=jnp.float32)
        m1 = y if m1 is None else jnp.maximum(m1, y)
    p1 = jnp.maximum(m1 + b1_ref[...], 0.0).astype(jnp.bfloat16)

    m2 = None
    for wref in (w2a_ref, w2b_ref, w2c_ref, w2d_ref):
        y = lax.dot_general(p1, wref[...], _DN,
                            preferred_element_type=jnp.float32)
        m2 = y if m2 is None else jnp.maximum(m2, y)
    p2 = jnp.maximum(m2 + b2_ref[...], 0.0).astype(jnp.bfloat16)

    h = lax.dot_general(p2, wf1_ref[...], _DN,
                        preferred_element_type=jnp.float32)
    h = jnp.maximum(h + bf1_ref[...], 0.0).astype(jnp.bfloat16)

    logits = lax.dot_general(h, wf2_ref[...], _DN,
                             preferred_element_type=jnp.float32) + bf2_ref[...]
    mx = jnp.max(logits, axis=-1, keepdims=True)
    s = logits - mx
    o_ref[...] = s - jnp.log(jnp.sum(jnp.exp(s), axis=-1, keepdims=True))


@jax.jit
def _forward(x_nchw, conv1_w, conv1_b, conv2_w, conv2_b,
             fc1_w, fc1_b, fc2_w, fc2_b):
    B = x_nchw.shape[0]
    bt = 512
    b_pad = ((B + bt - 1) // bt) * bt

    x = x_nchw.reshape(B, 784)
    if b_pad != B:
        x = jnp.pad(x, ((0, b_pad - B), (0, 0)))

    w1p = _build_w1(conv1_w)
    b1 = jnp.broadcast_to(conv1_b.astype(jnp.float32)[:, None],
                          (10, 144)).reshape(1, 1440)
    w2p = _build_w2(conv2_w)
    b2 = jnp.broadcast_to(conv2_b.astype(jnp.float32)[:, None],
                          (20, 16)).reshape(1, 320)
    wf1 = fc1_w.astype(jnp.bfloat16)
    bf1 = fc1_b.astype(jnp.float32).reshape(1, 50)
    wf2 = fc2_w.astype(jnp.bfloat16)
    bf2 = fc2_b.astype(jnp.float32).reshape(1, 10)

    flops = 2 * b_pad * (4 * 1440 * 784 + 4 * 320 * 1440 + 320 * 50 + 50 * 10)
    bytes_accessed = int(b_pad * 784 * 4 + 4 * (1440 * 784 + 320 * 1440) * 2
                         + b_pad * 10 * 4)
    whole = lambda shape: pl.BlockSpec(shape, lambda i: (0,) * len(shape))
    out = pl.pallas_call(
        _lenet_kernel,
        out_shape=jax.ShapeDtypeStruct((b_pad, 10), jnp.float32),
        grid=(b_pad // bt,),
        in_specs=[
            pl.BlockSpec((bt, 784), lambda i: (i, 0)),
            whole((1440, 784)), whole((1440, 784)),
            whole((1440, 784)), whole((1440, 784)),
            whole((1, 1440)),
            whole((320, 1440)), whole((320, 1440)),
            whole((320, 1440)), whole((320, 1440)),
            whole((1, 320)),
            whole((50, 320)), whole((1, 50)),
            whole((10, 50)), whole((1, 10)),
        ],
        out_specs=pl.BlockSpec((bt, 10), lambda i: (i, 0)),
        compiler_params=pltpu.CompilerParams(
            dimension_semantics=("parallel",),
            vmem_limit_bytes=56 << 20),
        cost_estimate=pl.CostEstimate(
            flops=flops, transcendentals=b_pad * 10,
            bytes_accessed=bytes_accessed),
    )(x, *w1p[:4], b1, *w2p[:4], b2, wf1, bf1, wf2, bf2)
    return out[:B]


def kernel(x_nchw, conv1_w, conv1_b, conv2_w, conv2_b,
           fc1_w, fc1_b, fc2_w, fc2_b):
    return _forward(x_nchw, conv1_w, conv1_b, conv2_w, conv2_b,
                    fc1_w, fc1_b, fc2_w, fc2_b)


# speedup vs baseline: 1.0159x; 1.0159x over previous
import jax
import jax.numpy as jnp
from jax import lax
from jax.experimental import pallas as pl
from jax.experimental.pallas import tpu as pltpu


def _tile_level(t, n_rows, row_len, modulus):
    pad = modulus - t.shape[-1]
    tp = jnp.pad(t, [(0, 0)] * (t.ndim - 1) + [(0, pad)])
    tiled = jnp.tile(tp[..., None, :],
                     tuple([1] * (t.ndim - 1)) + (n_rows, 1))
    flat = tiled.reshape(t.shape[:-1] + (n_rows * modulus,))
    flat = flat[..., : n_rows * row_len]
    return flat.reshape(t.shape[:-1] + (n_rows, row_len))


def _build_w1(conv1_w):
    k = conv1_w.reshape(10, 5, 5).astype(jnp.bfloat16)
    k = jnp.pad(k, ((0, 0), (0, 0), (0, 23))).reshape(10, 140)
    s = _tile_level(k, 2, 784, 785)
    s = _tile_level(s, 2, 784, 812)
    s = _tile_level(s, 12, 784, 840)
    s = _tile_level(s, 12, 784, 786)
    return [s[:, e, d].reshape(1440, 784)
            for e in range(2) for d in range(2)]


def _build_w2(conv2_w):
    k = conv2_w.astype(jnp.bfloat16)
    k = jnp.pad(k, ((0, 0), (0, 0), (0, 7), (0, 7)))
    k = k.reshape(20, 1440)
    s = _tile_level(k, 2, 1440, 1441)
    s = _tile_level(s, 2, 1440, 1452)
    s = _tile_level(s, 4, 1440, 1464)
    s = _tile_level(s, 4, 1440, 1442)
    return [s[:, e, d].reshape(320, 1440)
            for e in range(2) for d in range(2)]


_DN = (((1,), (1,)), ((), ()))


def _lenet_kernel(x_ref, w1a_ref, w1b_ref, w1c_ref, w1d_ref, b1_ref,
                  w2a_ref, w2b_ref, w2c_ref, w2d_ref, b2_ref,
                  wf1_ref, bf1_ref, wf2_ref, bf2_ref, o_ref):
    x = x_ref[...].astype(jnp.bfloat16)
    m1 = None
    for wref in (w1a_ref, w1b_ref, w1c_ref, w1d_ref):
        y = lax.dot_general(x, wref[...], _DN,
                            preferred_element_type=jnp.float32)
        m1 = y if m1 is None else jnp.maximum(m1, y)
    p1 = jnp.maximum(m1 + b1_ref[...], 0.0).astype(jnp.bfloat16)

    m2 = None
    for wref in (w2a_ref, w2b_ref, w2c_ref, w2d_ref):
        y = lax.dot_general(p1, wref[...], _DN,
                            preferred_element_type=jnp.float32)
        m2 = y if m2 is None else jnp.maximum(m2, y)
    p2 = jnp.maximum(m2 + b2_ref[...], 0.0).astype(jnp.bfloat16)

    h = lax.dot_general(p2, wf1_ref[...], _DN,
                        preferred_element_type=jnp.float32)
    h = jnp.maximum(h + bf1_ref[...], 0.0).astype(jnp.bfloat16)

    logits = lax.dot_general(h, wf2_ref[...], _DN,
                             preferred_element_type=jnp.float32) + bf2_ref[...]
    mx = jnp.max(logits, axis=-1, keepdims=True)
    s = logits - mx
    o_ref[...] = s - jnp.log(jnp.sum(jnp.exp(s), axis=-1, keepdims=True))


@jax.jit
def _forward(x_nchw, conv1_w, conv1_b, conv2_w, conv2_b,
             fc1_w, fc1_b, fc2_w, fc2_b):
    B = x_nchw.shape[0]
    bt = 1024
    b_pad = ((B + bt - 1) // bt) * bt

    x = x_nchw.reshape(B, 784)
    if b_pad != B:
        x = jnp.pad(x, ((0, b_pad - B), (0, 0)))

    w1p = _build_w1(conv1_w)
    b1 = jnp.broadcast_to(conv1_b.astype(jnp.float32)[:, None],
                          (10, 144)).reshape(1, 1440)
    w2p = _build_w2(conv2_w)
    b2 = jnp.broadcast_to(conv2_b.astype(jnp.float32)[:, None],
                          (20, 16)).reshape(1, 320)
    wf1 = fc1_w.astype(jnp.bfloat16)
    bf1 = fc1_b.astype(jnp.float32).reshape(1, 50)
    wf2 = fc2_w.astype(jnp.bfloat16)
    bf2 = fc2_b.astype(jnp.float32).reshape(1, 10)

    flops = 2 * b_pad * (4 * 1440 * 784 + 4 * 320 * 1440 + 320 * 50 + 50 * 10)
    bytes_accessed = int(b_pad * 784 * 4 + 4 * (1440 * 784 + 320 * 1440) * 2
                         + b_pad * 10 * 4)
    whole = lambda shape: pl.BlockSpec(shape, lambda i: (0,) * len(shape))
    out = pl.pallas_call(
        _lenet_kernel,
        out_shape=jax.ShapeDtypeStruct((b_pad, 10), jnp.float32),
        grid=(b_pad // bt,),
        in_specs=[
            pl.BlockSpec((bt, 784), lambda i: (i, 0)),
            whole((1440, 784)), whole((1440, 784)),
            whole((1440, 784)), whole((1440, 784)),
            whole((1, 1440)),
            whole((320, 1440)), whole((320, 1440)),
            whole((320, 1440)), whole((320, 1440)),
            whole((1, 320)),
            whole((50, 320)), whole((1, 50)),
            whole((10, 50)), whole((1, 10)),
        ],
        out_specs=pl.BlockSpec((bt, 10), lambda i: (i, 0)),
        compiler_params=pltpu.CompilerParams(
            dimension_semantics=("parallel",),
            vmem_limit_bytes=56 << 20),
        cost_estimate=pl.CostEstimate(
            flops=flops, transcendentals=b_pad * 10,
            bytes_accessed=bytes_accessed),
    )(x, *w1p[:4], b1, *w2p[:4], b2, wf1, bf1, wf2, bf2)
    return out[:B]


def kernel(x_nchw, conv1_w, conv1_b, conv2_w, conv2_b,
           fc1_w, fc1_b, fc2_w, fc2_b):
    return _forward(x_nchw, conv1_w, conv1_b, conv2_w, conv2_b,
                    fc1_w, fc1_b, fc2_w, fc2_b)
